# baseline (speedup 1.0000x reference)
import functools

import jax
import jax.numpy as jnp
from jax import lax
from jax.experimental import pallas as pl
from jax.experimental.pallas import tpu as pltpu
from jax.experimental.pallas import tpu_sc as plsc

N = 10000
D_IN = 128
E = 320000
NUM_GRAPHS = 64

NPAD = 10240
DUMMY = 10008
L = 128
NW = 32
NB = 80
EPAD = NW * NB * L
NBATCH = EPAD // L
NB0 = 32
NB1 = (NBATCH - 16 * NB0) // 16
NBMAX = max(NB0, NB1)
RPT = NPAD // 16
BS = 1280
GRID = NPAD // BS


def _sc_mesh():
    return plsc.VectorSubcoreMesh(core_axis_name="c", subcore_axis_name="s")


def _zero_vmem(buf, rows, cols):
    z = jnp.zeros((16,), jnp.float32)

    def body(i, _):
        for c in range(cols // 16):
            buf[i, pl.ds(c * 16, 16)] = z
        return 0

    lax.fori_loop(0, rows, body, 0)


def _zero_vmem2(buf):
    z = jnp.zeros((16,), jnp.float32).reshape(2, 8)

    def body(i, _):
        buf[pl.ds(i * 2, 2), :] = z
        return 0

    lax.fori_loop(0, L // 2, body, 0)


def _sc_degree(dst2d):
    @functools.partial(
        pl.kernel,
        out_type=jax.ShapeDtypeStruct((2, NPAD, 8), jnp.float32),
        mesh=_sc_mesh(),
        scratch_types=[
            pltpu.VMEM((NB, L), jnp.int32),
            pltpu.VMEM((L, 8), jnp.float32),
            pltpu.VMEM_SHARED((NPAD, 8), jnp.float32),
        ],
        compiler_params=pltpu.CompilerParams(use_tc_tiling_on_sc=False),
    )
    def k(dst_hbm, out_hbm, dstb, onesb, acc):
        cid = lax.axis_index("c")
        sid = lax.axis_index("s")
        wid = sid * 2 + cid

        _zero_vmem2(onesb)
        base = sid * RPT
        for c in range(RPT // L):
            pltpu.sync_copy(onesb, acc.at[pl.ds(base + c * L, L)])
        plsc.subcore_barrier()

        one = jnp.ones((16,), jnp.float32)

        def fill(i, _):
            onesb[pl.ds(i * 2, 2), :] = one.reshape(2, 8)
            return 0

        lax.fori_loop(0, L // 2, fill, 0)

        pltpu.sync_copy(dst_hbm.at[pl.ds(wid * NB, NB)], dstb)

        def body(j, _):
            pltpu.sync_copy(onesb, acc.at[dstb.at[j]], add=True)
            return 0

        lax.fori_loop(0, NB, body, 0)
        plsc.subcore_barrier()

        for c in range(RPT // L):
            pltpu.sync_copy(acc.at[pl.ds(base + c * L, L)], onesb)
            pltpu.sync_copy(onesb,
                            out_hbm.at[cid, pl.ds(base + c * L, L)])

    return k(dst2d)


def _sc_segsum(src2d, dst2d, h_pad, d):
    @functools.partial(
        pl.kernel,
        out_type=jax.ShapeDtypeStruct((2, NPAD, d), jnp.float32),
        mesh=_sc_mesh(),
        scratch_types=[
            pltpu.VMEM((NBMAX, L), jnp.int32),
            pltpu.VMEM((NBMAX, L), jnp.int32),
            [pltpu.VMEM((L, d), jnp.float32)] * 4,
            [pltpu.SemaphoreType.DMA] * 4,
            pltpu.VMEM_SHARED((NPAD, d), jnp.float32),
        ],
        compiler_params=pltpu.CompilerParams(use_tc_tiling_on_sc=False),
    )
    def k(src_hbm, dst_hbm, h_hbm, out_hbm,
          srcb, dstb, rows, sems, acc):
        cid = lax.axis_index("c")
        sid = lax.axis_index("s")
        wid = sid * 2 + cid

        _zero_vmem(rows[0], L, d)
        base = sid * RPT
        for c in range(RPT // L):
            pltpu.sync_copy(rows[0], acc.at[pl.ds(base + c * L, L)])
        plsc.subcore_barrier()

        nb = jnp.where(cid == 0, NB0, NB1)
        start = jnp.where(cid == 0, sid * NB0, 16 * NB0 + sid * NB1)
        pltpu.sync_copy(src_hbm.at[pl.ds(start, NBMAX)], srcb)
        pltpu.sync_copy(dst_hbm.at[pl.ds(start, NBMAX)], dstb)

        nbuf = 4
        for b in range(nbuf):
            pltpu.async_copy(h_hbm.at[srcb.at[b]], rows[b], sems[b])

        def body(i, _):
            for b in range(nbuf):
                j = i * nbuf + b
                pltpu.make_async_copy(
                    h_hbm.at[srcb.at[0]], rows[b], sems[b]).wait()
                pltpu.sync_copy(rows[b], acc.at[dstb.at[j]], add=True)

                @pl.when(j + nbuf < nb)
                def _():
                    pltpu.async_copy(
                        h_hbm.at[srcb.at[j + nbuf]], rows[b], sems[b])

            return 0

        lax.fori_loop(0, nb // nbuf, body, 0)
        plsc.subcore_barrier()

        for c in range(RPT // L):
            pltpu.sync_copy(acc.at[pl.ds(base + c * L, L)], rows[c % 4])
            pltpu.sync_copy(rows[c % 4],
                            out_hbm.at[cid, pl.ds(base + c * L, L)])

    return k(src2d, dst2d, h_pad)


def _tc_stats(x_pad):
    def body(x_ref, o_ref):
        i = pl.program_id(0)

        @pl.when(i == 0)
        def _():
            o_ref[...] = jnp.zeros_like(o_ref)

        xb = x_ref[...]
        s = jnp.sum(xb, axis=0, keepdims=True)
        s2 = jnp.sum(xb * xb, axis=0, keepdims=True)
        o_ref[...] += jnp.concatenate([s, s2], axis=0)

    return pl.pallas_call(
        body,
        grid=(GRID,),
        in_specs=[pl.BlockSpec((BS, D_IN), lambda i: (i, 0))],
        out_specs=pl.BlockSpec((2, D_IN), lambda i: (0, 0)),
        out_shape=jax.ShapeDtypeStruct((2, D_IN), jnp.float32),
    )(x_pad)


def _dis_block(d_ref):
    deg = d_ref[0] + d_ref[1]
    return lax.rsqrt(deg[:, 0:1] + 1.0)


def _tc_layer1(x_pad, stats, gamma2, beta2, w1, degp):
    def body(x_ref, st_ref, g_ref, b_ref, w_ref, d_ref, o_ref):
        xb = x_ref[...]
        mean = st_ref[0:1, :] * (1.0 / N)
        ex2 = st_ref[1:2, :] * (1.0 / N)
        inv = lax.rsqrt(ex2 - mean * mean + 1e-5)
        hb = (xb - mean) * (inv * g_ref[...]) + b_ref[...]
        dis = _dis_block(d_ref)
        o_ref[...] = jnp.dot(hb, w_ref[...],
                             preferred_element_type=jnp.float32) * dis

    return pl.pallas_call(
        body,
        grid=(GRID,),
        in_specs=[
            pl.BlockSpec((BS, D_IN), lambda i: (i, 0)),
            pl.BlockSpec((2, D_IN), lambda i: (0, 0)),
            pl.BlockSpec((1, D_IN), lambda i: (0, 0)),
            pl.BlockSpec((1, D_IN), lambda i: (0, 0)),
            pl.BlockSpec((D_IN, 64), lambda i: (0, 0)),
            pl.BlockSpec((2, BS, 8), lambda i: (0, i, 0)),
        ],
        out_specs=pl.BlockSpec((BS, 64), lambda i: (i, 0)),
        out_shape=jax.ShapeDtypeStruct((NPAD, 64), jnp.float32),
    )(x_pad, stats, gamma2, beta2, w1, degp)


def _tc_layer(s_parts, hprev, degp, bias2, w, dout):
    din = hprev.shape[1]

    def body(s_ref, h_ref, d_ref, b_ref, w_ref, o_ref):
        dis = _dis_block(d_ref)
        t = (s_ref[0] + s_ref[1] + h_ref[...]) * dis + b_ref[...]
        t = jnp.maximum(t, 0.0)
        o_ref[...] = jnp.dot(t, w_ref[...],
                             preferred_element_type=jnp.float32) * dis

    return pl.pallas_call(
        body,
        grid=(GRID,),
        in_specs=[
            pl.BlockSpec((2, BS, din), lambda i: (0, i, 0)),
            pl.BlockSpec((BS, din), lambda i: (i, 0)),
            pl.BlockSpec((2, BS, 8), lambda i: (0, i, 0)),
            pl.BlockSpec((1, din), lambda i: (0, 0)),
            pl.BlockSpec((din, dout), lambda i: (0, 0)),
        ],
        out_specs=pl.BlockSpec((BS, dout), lambda i: (i, 0)),
        out_shape=jax.ShapeDtypeStruct((NPAD, dout), jnp.float32),
    )(s_parts, hprev, degp, bias2, w)


def _tc_final(s_parts, h3, degp, b32, batch3d, wc1, bc1r, wc2, bc2r):
    def body(s_ref, h_ref, d_ref, b_ref, bt_ref, w1_ref, b1_ref,
             w2_ref, b2_ref, o_ref, pooled, cnt):
        i = pl.program_id(0)

        @pl.when(i == 0)
        def _():
            pooled[...] = jnp.zeros_like(pooled)
            cnt[...] = jnp.zeros_like(cnt)

        dis = _dis_block(d_ref)
        hb = (s_ref[0] + s_ref[1] + h_ref[...]) * dis + b_ref[...]
        hb = jnp.maximum(hb, 0.0)
        bt = bt_ref[0]
        ohT = (lax.broadcasted_iota(jnp.int32, (NUM_GRAPHS, BS), 0)
               == bt).astype(jnp.float32)
        pooled[...] += jnp.dot(ohT, hb, preferred_element_type=jnp.float32)
        cnt[...] += jnp.sum(ohT, axis=1, keepdims=True)

        @pl.when(i == GRID - 1)
        def _():
            pm = pooled[...] / jnp.maximum(cnt[:, 0:1], 1.0)
            z1 = jnp.maximum(
                jnp.dot(pm, w1_ref[...], preferred_element_type=jnp.float32)
                + b1_ref[...], 0.0)
            o_ref[...] = jnp.dot(
                z1, w2_ref[...], preferred_element_type=jnp.float32) + b2_ref[...]

    return pl.pallas_call(
        body,
        grid=(GRID,),
        in_specs=[
            pl.BlockSpec((2, BS, 32), lambda i: (0, i, 0)),
            pl.BlockSpec((BS, 32), lambda i: (i, 0)),
            pl.BlockSpec((2, BS, 8), lambda i: (0, i, 0)),
            pl.BlockSpec((1, 32), lambda i: (0, 0)),
            pl.BlockSpec((1, 1, BS), lambda i: (i, 0, 0)),
            pl.BlockSpec((32, 16), lambda i: (0, 0)),
            pl.BlockSpec((1, 16), lambda i: (0, 0)),
            pl.BlockSpec((16, 2), lambda i: (0, 0)),
            pl.BlockSpec((1, 2), lambda i: (0, 0)),
        ],
        out_specs=pl.BlockSpec((NUM_GRAPHS, 2), lambda i: (0, 0)),
        out_shape=jax.ShapeDtypeStruct((NUM_GRAPHS, 2), jnp.float32),
        scratch_shapes=[
            pltpu.VMEM((NUM_GRAPHS, 32), jnp.float32),
            pltpu.VMEM((NUM_GRAPHS, 8), jnp.float32),
        ],
    )(s_parts, h3, degp, b32, batch3d, wc1, bc1r, wc2, bc2r)


def kernel(x, edge_index, batch, gamma, beta,
           W1, b1, W2, b2, W3, b3, Wc1, bc1, Wc2, bc2):
    x_pad = jnp.pad(x, ((0, NPAD - N), (0, 0)))
    pad_e = jnp.full((EPAD - E,), DUMMY, jnp.int32)
    src2d = jnp.concatenate([edge_index[0], pad_e]).reshape(EPAD // L, L)
    dst2d = jnp.concatenate([edge_index[1], pad_e]).reshape(EPAD // L, L)
    batch3d = jnp.pad(batch, (0, NPAD - N),
                      constant_values=NUM_GRAPHS).reshape(GRID, 1, BS)

    degp = _sc_degree(dst2d)
    stats = _tc_stats(x_pad)
    h1 = _tc_layer1(x_pad, stats, gamma.reshape(1, D_IN),
                    beta.reshape(1, D_IN), W1, degp)
    s1 = _sc_segsum(src2d, dst2d, h1, 64)
    h2 = _tc_layer(s1, h1, degp, b1.reshape(1, 64), W2, 64)
    s2 = _sc_segsum(src2d, dst2d, h2, 64)
    h3 = _tc_layer(s2, h2, degp, b2.reshape(1, 64), W3, 32)
    s3 = _sc_segsum(src2d, dst2d, h3, 32)
    return _tc_final(s3, h3, degp, b3.reshape(1, 32), batch3d,
                     Wc1, bc1.reshape(1, 16), Wc2, bc2.reshape(1, 2))


# speedup vs baseline: 1.6037x; 1.6037x over previous
import functools

import jax
import jax.numpy as jnp
from jax import lax
from jax.experimental import pallas as pl
from jax.experimental.pallas import tpu as pltpu
from jax.experimental.pallas import tpu_sc as plsc

N = 10000
D_IN = 128
E = 320000
NUM_GRAPHS = 64

NPAD = 10240
DUMMY = 10008
L = 128
NW = 32
NB = 80
EPAD = NW * NB * L
RPT = NPAD // 16
BS = 1280
GRID = NPAD // BS


def _sc_mesh():
    return plsc.VectorSubcoreMesh(core_axis_name="c", subcore_axis_name="s")


def _zero_vmem(buf, rows, cols):
    z = jnp.zeros((16,), jnp.float32)

    def body(i, _):
        for c in range(cols // 16):
            buf[i, pl.ds(c * 16, 16)] = z
        return 0

    lax.fori_loop(0, rows, body, 0)


def _sc_segsum(src2d, dst2d, h_pad):
    d = 64

    @functools.partial(
        pl.kernel,
        out_type=jax.ShapeDtypeStruct((2, NPAD, d), jnp.float32),
        mesh=_sc_mesh(),
        scratch_types=[
            pltpu.VMEM((NB // 2, L), jnp.int32),
            pltpu.VMEM((NB // 2, L), jnp.int32),
            [pltpu.VMEM((L, d), jnp.float32)] * 4,
            [pltpu.SemaphoreType.DMA] * 4,
            pltpu.VMEM_SHARED((NPAD, d), jnp.float32),
            pltpu.VMEM_SHARED((NPAD, d), jnp.float32),
        ],
        compiler_params=pltpu.CompilerParams(use_tc_tiling_on_sc=False),
    )
    def k(src_hbm, dst_hbm, h_hbm, out_hbm,
          srcb, dstb, rows, sems, h_sp, acc):
        cid = lax.axis_index("c")
        sid = lax.axis_index("s")
        wid = sid * 2 + cid

        _zero_vmem(rows[0], L, d)
        base = sid * RPT
        for c in range(RPT // L):
            pltpu.sync_copy(rows[0], acc.at[pl.ds(base + c * L, L)])
            pltpu.sync_copy(h_hbm.at[pl.ds(base + c * L, L)], rows[1])
            pltpu.sync_copy(rows[1], h_sp.at[pl.ds(base + c * L, L)])
        plsc.subcore_barrier()

        nbuf = 4
        nbh = NB // 2

        def body(i, _):
            for b in range(nbuf):
                j = i * nbuf + b
                pltpu.make_async_copy(
                    h_sp.at[srcb.at[0]], rows[b], sems[b]).wait()
                pltpu.sync_copy(rows[b], acc.at[dstb.at[j]], add=True)

                @pl.when(j + nbuf < nbh)
                def _():
                    pltpu.async_copy(
                        h_sp.at[srcb.at[j + nbuf]], rows[b], sems[b])

            return 0

        for half in range(2):
            pltpu.sync_copy(
                src_hbm.at[pl.ds(wid * NB + half * nbh, nbh)], srcb)
            pltpu.sync_copy(
                dst_hbm.at[pl.ds(wid * NB + half * nbh, nbh)], dstb)
            for b in range(nbuf):
                pltpu.async_copy(h_sp.at[srcb.at[b]], rows[b], sems[b])
            lax.fori_loop(0, nbh // nbuf, body, 0)

        plsc.subcore_barrier()

        for c in range(RPT // L):
            pltpu.sync_copy(acc.at[pl.ds(base + c * L, L)], rows[c % 4])
            pltpu.sync_copy(rows[c % 4],
                            out_hbm.at[cid, pl.ds(base + c * L, L)])

    return k(src2d, dst2d, h_pad)


def _tc_stats(x_pad):
    def body(x_ref, o_ref):
        i = pl.program_id(0)

        @pl.when(i == 0)
        def _():
            o_ref[...] = jnp.zeros_like(o_ref)

        xb = x_ref[...]
        s = jnp.sum(xb, axis=0, keepdims=True)
        s2 = jnp.sum(xb * xb, axis=0, keepdims=True)
        o_ref[...] += jnp.concatenate([s, s2], axis=0)

    return pl.pallas_call(
        body,
        grid=(GRID,),
        in_specs=[pl.BlockSpec((BS, D_IN), lambda i: (i, 0))],
        out_specs=pl.BlockSpec((2, D_IN), lambda i: (0, 0)),
        out_shape=jax.ShapeDtypeStruct((2, D_IN), jnp.float32),
    )(x_pad)


def _dis_block(d_ref):
    deg = d_ref[0] + d_ref[1]
    return lax.rsqrt(deg[:, 0:1] + 1.0)


def _tc_layer1(x_pad, stats, gamma2, beta2, w1, degp):
    def body(x_ref, st_ref, g_ref, b_ref, w_ref, d_ref, o_ref):
        xb = x_ref[...]
        mean = st_ref[0:1, :] * (1.0 / N)
        ex2 = st_ref[1:2, :] * (1.0 / N)
        inv = lax.rsqrt(ex2 - mean * mean + 1e-5)
        hb = (xb - mean) * (inv * g_ref[...]) + b_ref[...]
        dis = _dis_block(d_ref)
        o_ref[...] = jnp.dot(hb, w_ref[...],
                             preferred_element_type=jnp.float32) * dis

    return pl.pallas_call(
        body,
        grid=(GRID,),
        in_specs=[
            pl.BlockSpec((BS, D_IN), lambda i: (i, 0)),
            pl.BlockSpec((2, D_IN), lambda i: (0, 0)),
            pl.BlockSpec((1, D_IN), lambda i: (0, 0)),
            pl.BlockSpec((1, D_IN), lambda i: (0, 0)),
            pl.BlockSpec((D_IN, 64), lambda i: (0, 0)),
            pl.BlockSpec((2, BS, 64), lambda i: (0, i, 0)),
        ],
        out_specs=pl.BlockSpec((BS, 64), lambda i: (i, 0)),
        out_shape=jax.ShapeDtypeStruct((NPAD, 64), jnp.float32),
    )(x_pad, stats, gamma2, beta2, w1, degp)


def _tc_layer(s_parts, hprev, degp, bias2, w, dout):
    din = hprev.shape[1]

    def body(s_ref, h_ref, d_ref, b_ref, w_ref, o_ref):
        dis = _dis_block(d_ref)
        t = (s_ref[0] + s_ref[1] + h_ref[...]) * dis + b_ref[...]
        t = jnp.maximum(t, 0.0)
        o_ref[...] = jnp.dot(t, w_ref[...],
                             preferred_element_type=jnp.float32) * dis

    return pl.pallas_call(
        body,
        grid=(GRID,),
        in_specs=[
            pl.BlockSpec((2, BS, din), lambda i: (0, i, 0)),
            pl.BlockSpec((BS, din), lambda i: (i, 0)),
            pl.BlockSpec((2, BS, 64), lambda i: (0, i, 0)),
            pl.BlockSpec((1, din), lambda i: (0, 0)),
            pl.BlockSpec((din, dout), lambda i: (0, 0)),
        ],
        out_specs=pl.BlockSpec((BS, dout), lambda i: (i, 0)),
        out_shape=jax.ShapeDtypeStruct((NPAD, dout), jnp.float32),
    )(s_parts, hprev, degp, bias2, w)


def _tc_final(s_parts, h3, degp, b32, batch3d, wc1, bc1r, wc2, bc2r):
    def body(s_ref, h_ref, d_ref, b_ref, bt_ref, w1_ref, b1_ref,
             w2_ref, b2_ref, o_ref, pooled, cnt):
        i = pl.program_id(0)

        @pl.when(i == 0)
        def _():
            pooled[...] = jnp.zeros_like(pooled)
            cnt[...] = jnp.zeros_like(cnt)

        dis = _dis_block(d_ref)
        hb = (s_ref[0] + s_ref[1] + h_ref[...]) * dis + b_ref[...]
        hb = jnp.maximum(hb, 0.0)[:, :32]
        bt = bt_ref[0]
        ohT = (lax.broadcasted_iota(jnp.int32, (NUM_GRAPHS, BS), 0)
               == bt).astype(jnp.float32)
        pooled[...] += jnp.dot(ohT, hb, preferred_element_type=jnp.float32)
        cnt[...] += jnp.sum(ohT, axis=1, keepdims=True)

        @pl.when(i == GRID - 1)
        def _():
            pm = pooled[...] / jnp.maximum(cnt[:, 0:1], 1.0)
            z1 = jnp.maximum(
                jnp.dot(pm, w1_ref[...], preferred_element_type=jnp.float32)
                + b1_ref[...], 0.0)
            o_ref[...] = jnp.dot(
                z1, w2_ref[...], preferred_element_type=jnp.float32) + b2_ref[...]

    return pl.pallas_call(
        body,
        grid=(GRID,),
        in_specs=[
            pl.BlockSpec((2, BS, 64), lambda i: (0, i, 0)),
            pl.BlockSpec((BS, 64), lambda i: (i, 0)),
            pl.BlockSpec((2, BS, 64), lambda i: (0, i, 0)),
            pl.BlockSpec((1, 64), lambda i: (0, 0)),
            pl.BlockSpec((1, 1, BS), lambda i: (i, 0, 0)),
            pl.BlockSpec((32, 16), lambda i: (0, 0)),
            pl.BlockSpec((1, 16), lambda i: (0, 0)),
            pl.BlockSpec((16, 2), lambda i: (0, 0)),
            pl.BlockSpec((1, 2), lambda i: (0, 0)),
        ],
        out_specs=pl.BlockSpec((NUM_GRAPHS, 2), lambda i: (0, 0)),
        out_shape=jax.ShapeDtypeStruct((NUM_GRAPHS, 2), jnp.float32),
        scratch_shapes=[
            pltpu.VMEM((NUM_GRAPHS, 32), jnp.float32),
            pltpu.VMEM((NUM_GRAPHS, 8), jnp.float32),
        ],
    )(s_parts, h3, degp, b32, batch3d, wc1, bc1r, wc2, bc2r)


def kernel(x, edge_index, batch, gamma, beta,
           W1, b1, W2, b2, W3, b3, Wc1, bc1, Wc2, bc2):
    x_pad = jnp.pad(x, ((0, NPAD - N), (0, 0)))
    pad_e = jnp.full((EPAD - E,), DUMMY, jnp.int32)
    src2d = jnp.concatenate([edge_index[0], pad_e]).reshape(EPAD // L, L)
    dst2d = jnp.concatenate([edge_index[1], pad_e]).reshape(EPAD // L, L)
    batch3d = jnp.pad(batch, (0, NPAD - N),
                      constant_values=NUM_GRAPHS).reshape(GRID, 1, BS)

    ones = jnp.ones((NPAD, 64), jnp.float32)
    degp = _sc_segsum(src2d, dst2d, ones)
    stats = _tc_stats(x_pad)
    h1 = _tc_layer1(x_pad, stats, gamma.reshape(1, D_IN),
                    beta.reshape(1, D_IN), W1, degp)
    s1 = _sc_segsum(src2d, dst2d, h1)
    h2 = _tc_layer(s1, h1, degp, b1.reshape(1, 64), W2, 64)
    s2 = _sc_segsum(src2d, dst2d, h2)
    w3p = jnp.pad(W3, ((0, 0), (0, 64 - W3.shape[1])))
    h3 = _tc_layer(s2, h2, degp, b2.reshape(1, 64), w3p, 64)
    s3 = _sc_segsum(src2d, dst2d, h3)
    b3p = jnp.pad(b3, (0, 64 - b3.shape[0])).reshape(1, 64)
    return _tc_final(s3, h3, degp, b3p, batch3d,
                     Wc1, bc1.reshape(1, 16), Wc2, bc2.reshape(1, 2))


# speedup vs baseline: 1.7951x; 1.1194x over previous
import functools

import jax
import jax.numpy as jnp
from jax import lax
from jax.experimental import pallas as pl
from jax.experimental.pallas import tpu as pltpu
from jax.experimental.pallas import tpu_sc as plsc

N = 10000
D_IN = 128
E = 320000
NUM_GRAPHS = 64

NPAD = 10240
DUMMY = 10008
L = 128
NW = 32
NB = 80
EPAD = NW * NB * L
RPT = NPAD // 16
BS = 1280
GRID = NPAD // BS


def _sc_mesh():
    return plsc.VectorSubcoreMesh(core_axis_name="c", subcore_axis_name="s")


def _zero_vmem(buf, rows, cols):
    z = jnp.zeros((16,), jnp.float32)

    def body(i, _):
        for c in range(cols // 16):
            buf[i, pl.ds(c * 16, 16)] = z
        return 0

    lax.fori_loop(0, rows, body, 0)


def _sc_segsum(src2d, dst2d, h_pad):
    d = 64

    @functools.partial(
        pl.kernel,
        out_type=jax.ShapeDtypeStruct((2, NPAD, d), jnp.float32),
        mesh=_sc_mesh(),
        scratch_types=[
            pltpu.VMEM((NB // 2, L), jnp.int32),
            pltpu.VMEM((NB // 2, L), jnp.int32),
            [pltpu.VMEM((L, d), jnp.float32)] * 4,
            [pltpu.SemaphoreType.DMA] * 4,
            [pltpu.SemaphoreType.DMA] * 4,
            pltpu.VMEM_SHARED((NPAD, d), jnp.float32),
            pltpu.VMEM_SHARED((NPAD, d), jnp.float32),
        ],
        compiler_params=pltpu.CompilerParams(use_tc_tiling_on_sc=False),
    )
    def k(src_hbm, dst_hbm, h_hbm, out_hbm,
          srcb, dstb, rows, sems, ssems, h_sp, acc):
        cid = lax.axis_index("c")
        sid = lax.axis_index("s")
        wid = sid * 2 + cid

        _zero_vmem(rows[0], L, d)
        base = sid * RPT
        for c in range(RPT // L):
            pltpu.sync_copy(rows[0], acc.at[pl.ds(base + c * L, L)])
            pltpu.sync_copy(h_hbm.at[pl.ds(base + c * L, L)], rows[1])
            pltpu.sync_copy(rows[1], h_sp.at[pl.ds(base + c * L, L)])
        plsc.subcore_barrier()

        nbuf = 4
        nbh = NB // 2

        def wait_g(b):
            pltpu.make_async_copy(h_sp.at[srcb.at[0]], rows[b], sems[b]).wait()

        def wait_s(b):
            pltpu.make_async_copy(rows[b], acc.at[dstb.at[0]], ssems[b]).wait()

        def body(i, _):
            for k_ in range(nbuf):
                j = i * nbuf + k_
                bn = (k_ + 2) % nbuf

                @pl.when(j >= 2)
                def _():
                    wait_s(bn)

                @pl.when(j + 2 < nbh)
                def _():
                    pltpu.async_copy(
                        h_sp.at[srcb.at[j + 2]], rows[bn], sems[bn])

                wait_g(k_)
                pltpu.async_copy(rows[k_], acc.at[dstb.at[j]],
                                 ssems[k_], add=True)
            return 0

        for half in range(2):
            pltpu.sync_copy(
                src_hbm.at[pl.ds(wid * NB + half * nbh, nbh)], srcb)
            pltpu.sync_copy(
                dst_hbm.at[pl.ds(wid * NB + half * nbh, nbh)], dstb)
            for b in range(2):
                pltpu.async_copy(h_sp.at[srcb.at[b]], rows[b], sems[b])
            lax.fori_loop(0, nbh // nbuf, body, 0)
            wait_s((nbh - 2) % nbuf)
            wait_s((nbh - 1) % nbuf)

        plsc.subcore_barrier()

        for c in range(RPT // L):
            pltpu.sync_copy(acc.at[pl.ds(base + c * L, L)], rows[c % 4])
            pltpu.sync_copy(rows[c % 4],
                            out_hbm.at[cid, pl.ds(base + c * L, L)])

    return k(src2d, dst2d, h_pad)


def _tc_stats(x_pad):
    def body(x_ref, o_ref):
        i = pl.program_id(0)

        @pl.when(i == 0)
        def _():
            o_ref[...] = jnp.zeros_like(o_ref)

        xb = x_ref[...]
        s = jnp.sum(xb, axis=0, keepdims=True)
        s2 = jnp.sum(xb * xb, axis=0, keepdims=True)
        o_ref[...] += jnp.concatenate([s, s2], axis=0)

    return pl.pallas_call(
        body,
        grid=(GRID,),
        in_specs=[pl.BlockSpec((BS, D_IN), lambda i: (i, 0))],
        out_specs=pl.BlockSpec((2, D_IN), lambda i: (0, 0)),
        out_shape=jax.ShapeDtypeStruct((2, D_IN), jnp.float32),
    )(x_pad)


def _dis_block(d_ref):
    deg = d_ref[0] + d_ref[1]
    return lax.rsqrt(deg[:, 0:1] + 1.0)


def _tc_layer1(x_pad, stats, gamma2, beta2, w1, degp):
    def body(x_ref, st_ref, g_ref, b_ref, w_ref, d_ref, o_ref):
        xb = x_ref[...]
        mean = st_ref[0:1, :] * (1.0 / N)
        ex2 = st_ref[1:2, :] * (1.0 / N)
        inv = lax.rsqrt(ex2 - mean * mean + 1e-5)
        hb = (xb - mean) * (inv * g_ref[...]) + b_ref[...]
        dis = _dis_block(d_ref)
        o_ref[...] = jnp.dot(hb, w_ref[...],
                             preferred_element_type=jnp.float32) * dis

    return pl.pallas_call(
        body,
        grid=(GRID,),
        in_specs=[
            pl.BlockSpec((BS, D_IN), lambda i: (i, 0)),
            pl.BlockSpec((2, D_IN), lambda i: (0, 0)),
            pl.BlockSpec((1, D_IN), lambda i: (0, 0)),
            pl.BlockSpec((1, D_IN), lambda i: (0, 0)),
            pl.BlockSpec((D_IN, 64), lambda i: (0, 0)),
            pl.BlockSpec((2, BS, 64), lambda i: (0, i, 0)),
        ],
        out_specs=pl.BlockSpec((BS, 64), lambda i: (i, 0)),
        out_shape=jax.ShapeDtypeStruct((NPAD, 64), jnp.float32),
    )(x_pad, stats, gamma2, beta2, w1, degp)


def _tc_layer(s_parts, hprev, degp, bias2, w, dout):
    din = hprev.shape[1]

    def body(s_ref, h_ref, d_ref, b_ref, w_ref, o_ref):
        dis = _dis_block(d_ref)
        t = (s_ref[0] + s_ref[1] + h_ref[...]) * dis + b_ref[...]
        t = jnp.maximum(t, 0.0)
        o_ref[...] = jnp.dot(t, w_ref[...],
                             preferred_element_type=jnp.float32) * dis

    return pl.pallas_call(
        body,
        grid=(GRID,),
        in_specs=[
            pl.BlockSpec((2, BS, din), lambda i: (0, i, 0)),
            pl.BlockSpec((BS, din), lambda i: (i, 0)),
            pl.BlockSpec((2, BS, 64), lambda i: (0, i, 0)),
            pl.BlockSpec((1, din), lambda i: (0, 0)),
            pl.BlockSpec((din, dout), lambda i: (0, 0)),
        ],
        out_specs=pl.BlockSpec((BS, dout), lambda i: (i, 0)),
        out_shape=jax.ShapeDtypeStruct((NPAD, dout), jnp.float32),
    )(s_parts, hprev, degp, bias2, w)


def _tc_final(s_parts, h3, degp, b32, batch3d, wc1, bc1r, wc2, bc2r):
    def body(s_ref, h_ref, d_ref, b_ref, bt_ref, w1_ref, b1_ref,
             w2_ref, b2_ref, o_ref, pooled, cnt):
        i = pl.program_id(0)

        @pl.when(i == 0)
        def _():
            pooled[...] = jnp.zeros_like(pooled)
            cnt[...] = jnp.zeros_like(cnt)

        dis = _dis_block(d_ref)
        hb = (s_ref[0] + s_ref[1] + h_ref[...]) * dis + b_ref[...]
        hb = jnp.maximum(hb, 0.0)[:, :32]
        bt = bt_ref[0]
        ohT = (lax.broadcasted_iota(jnp.int32, (NUM_GRAPHS, BS), 0)
               == bt).astype(jnp.float32)
        pooled[...] += jnp.dot(ohT, hb, preferred_element_type=jnp.float32)
        cnt[...] += jnp.sum(ohT, axis=1, keepdims=True)

        @pl.when(i == GRID - 1)
        def _():
            pm = pooled[...] / jnp.maximum(cnt[:, 0:1], 1.0)
            z1 = jnp.maximum(
                jnp.dot(pm, w1_ref[...], preferred_element_type=jnp.float32)
                + b1_ref[...], 0.0)
            o_ref[...] = jnp.dot(
                z1, w2_ref[...], preferred_element_type=jnp.float32) + b2_ref[...]

    return pl.pallas_call(
        body,
        grid=(GRID,),
        in_specs=[
            pl.BlockSpec((2, BS, 64), lambda i: (0, i, 0)),
            pl.BlockSpec((BS, 64), lambda i: (i, 0)),
            pl.BlockSpec((2, BS, 64), lambda i: (0, i, 0)),
            pl.BlockSpec((1, 64), lambda i: (0, 0)),
            pl.BlockSpec((1, 1, BS), lambda i: (i, 0, 0)),
            pl.BlockSpec((32, 16), lambda i: (0, 0)),
            pl.BlockSpec((1, 16), lambda i: (0, 0)),
            pl.BlockSpec((16, 2), lambda i: (0, 0)),
            pl.BlockSpec((1, 2), lambda i: (0, 0)),
        ],
        out_specs=pl.BlockSpec((NUM_GRAPHS, 2), lambda i: (0, 0)),
        out_shape=jax.ShapeDtypeStruct((NUM_GRAPHS, 2), jnp.float32),
        scratch_shapes=[
            pltpu.VMEM((NUM_GRAPHS, 32), jnp.float32),
            pltpu.VMEM((NUM_GRAPHS, 8), jnp.float32),
        ],
    )(s_parts, h3, degp, b32, batch3d, wc1, bc1r, wc2, bc2r)


def kernel(x, edge_index, batch, gamma, beta,
           W1, b1, W2, b2, W3, b3, Wc1, bc1, Wc2, bc2):
    x_pad = jnp.pad(x, ((0, NPAD - N), (0, 0)))
    pad_e = jnp.full((EPAD - E,), DUMMY, jnp.int32)
    src2d = jnp.concatenate([edge_index[0], pad_e]).reshape(EPAD // L, L)
    dst2d = jnp.concatenate([edge_index[1], pad_e]).reshape(EPAD // L, L)
    batch3d = jnp.pad(batch, (0, NPAD - N),
                      constant_values=NUM_GRAPHS).reshape(GRID, 1, BS)

    ones = jnp.ones((NPAD, 64), jnp.float32)
    degp = _sc_segsum(src2d, dst2d, ones)
    stats = _tc_stats(x_pad)
    h1 = _tc_layer1(x_pad, stats, gamma.reshape(1, D_IN),
                    beta.reshape(1, D_IN), W1, degp)
    s1 = _sc_segsum(src2d, dst2d, h1)
    h2 = _tc_layer(s1, h1, degp, b1.reshape(1, 64), W2, 64)
    s2 = _sc_segsum(src2d, dst2d, h2)
    w3p = jnp.pad(W3, ((0, 0), (0, 64 - W3.shape[1])))
    h3 = _tc_layer(s2, h2, degp, b2.reshape(1, 64), w3p, 64)
    s3 = _sc_segsum(src2d, dst2d, h3)
    b3p = jnp.pad(b3, (0, 64 - b3.shape[0])).reshape(1, 64)
    return _tc_final(s3, h3, degp, b3p, batch3d,
                     Wc1, bc1.reshape(1, 16), Wc2, bc2.reshape(1, 2))
